# Initial kernel scaffold; baseline (speedup 1.0000x reference)
import jax
import jax.numpy as jnp
from jax.experimental import pallas as pl


def kernel(sto_x, sto_edge_index, sto_weight, sto_batch, weight, W1, b1, W2, b2, Wih, Whh, bih, bhh, mean_W, mean_b, P1, pb1, P2, pb2, P3, pb3):
    raise NotImplementedError("write your pallas kernel here")


# speedup vs baseline: 7.7234x; 7.7234x over previous
import functools

import jax
import jax.numpy as jnp
from jax import lax
from jax.experimental import pallas as pl
from jax.experimental.pallas import tpu as pltpu
from jax.experimental.pallas import tpu_sc as plsc

_N = 10000
_E = 320000
_D = 128
_H = 128
_V = 10000
_B = 64

_NPAD = 10240
_ROWS_PER_TILE = _NPAD // 32
_EC = 128
_NCHUNK = 80
_EPT = _EC * _NCHUNK
_EPAD = _EPT * 32
_DEG_PT = _E // 32

_mesh = plsc.VectorSubcoreMesh(core_axis_name="c", subcore_axis_name="s")


@functools.partial(
    pl.kernel,
    out_type=(
        jax.ShapeDtypeStruct((_NPAD, _D), jnp.float32),
        jax.ShapeDtypeStruct((2, _NPAD), jnp.float32),
    ),
    mesh=_mesh,
    scratch_types=[
        pltpu.VMEM((4, 80), jnp.int32),
        pltpu.VMEM((_ROWS_PER_TILE, _D), jnp.float32),
        pltpu.VMEM((_DEG_PT,), jnp.int32),
        pltpu.VMEM((_NPAD,), jnp.float32),
        pltpu.VMEM((_NPAD // 16,), jnp.float32),
        pltpu.VMEM((_NPAD // 16,), jnp.float32),
        pltpu.VMEM_SHARED((16, _NPAD), jnp.float32),
        pltpu.SemaphoreType.DMA,
    ],
    compiler_params=pltpu.CompilerParams(needs_layout_passes=False),
)
def _sc_gather_deg(stox_hbm, dstd_hbm, zeros1_hbm, table_hbm,
                   x_hbm, deg_hbm,
                   idx_v, rows_v, dstl_v, degl_v, tmp_v, dacc_v, deg_st, sem):
    c = lax.axis_index("c")
    s = lax.axis_index("s")
    wid = c * 16 + s

    pltpu.sync_copy(stox_hbm.at[wid], idx_v)
    for r in range(4):
        for k in range(5):
            v = idx_v[r, pl.ds(k * 16, 16)]
            v = jnp.where(v == 0, _V - 1, v - 1)
            idx_v[r, pl.ds(k * 16, 16)] = v
    for r in range(4):
        pltpu.async_copy(table_hbm.at[idx_v.at[r]],
                         rows_v.at[pl.ds(r * 80, 80)], sem).wait()
    pltpu.sync_copy(rows_v, x_hbm.at[pl.ds(wid * _ROWS_PER_TILE, _ROWS_PER_TILE)])

    pltpu.sync_copy(dstd_hbm.at[wid], dstl_v)
    pltpu.sync_copy(zeros1_hbm, degl_v)
    ones = jnp.full((16,), 1.0, jnp.float32)

    def deg_body(k, _):
        dv = dstl_v[pl.ds(k * 16, 16)]
        plsc.addupdate_scatter(degl_v, [dv], ones)
        return 0

    lax.fori_loop(0, _DEG_PT // 16, deg_body, 0)

    pltpu.sync_copy(degl_v, deg_st.at[s])
    plsc.subcore_barrier()
    nsl = _NPAD // 16
    rslice = pl.ds(s * nsl, nsl)
    pltpu.sync_copy(deg_st.at[0, rslice], dacc_v)
    for slot in range(1, 16):
        pltpu.sync_copy(deg_st.at[slot, rslice], tmp_v)

        def add_body(k, _):
            dacc_v[pl.ds(k * 16, 16)] = (dacc_v[pl.ds(k * 16, 16)]
                                         + tmp_v[pl.ds(k * 16, 16)])
            return 0

        lax.fori_loop(0, nsl // 16, add_body, 0)
    pltpu.sync_copy(dacc_v, deg_hbm.at[c, rslice])


@functools.partial(
    pl.kernel,
    out_type=jax.ShapeDtypeStruct((2, _NPAD, _D), jnp.float32),
    mesh=_mesh,
    scratch_types=[
        pltpu.VMEM((_NCHUNK, _EC), jnp.int32),
        pltpu.VMEM((_NCHUNK, _EC), jnp.int32),
        pltpu.VMEM((_EC, _D), jnp.float32),
        pltpu.VMEM_SHARED((_NPAD, _D), jnp.float32),
        pltpu.SemaphoreType.DMA,
    ],
    compiler_params=pltpu.CompilerParams(needs_layout_passes=False),
)
def _sc_edge_pass(src_hbm, dst_hbm, zeros2_hbm, hp_hbm,
                  acc_hbm,
                  src_v, dst_v, rows_v, acc_sh, sem):
    c = lax.axis_index("c")
    s = lax.axis_index("s")
    rpt = _NPAD // 16

    pltpu.sync_copy(src_hbm.at[c, s], src_v)
    pltpu.sync_copy(dst_hbm.at[c, s], dst_v)
    pltpu.sync_copy(zeros2_hbm.at[pl.ds(s * rpt, rpt)],
                    acc_sh.at[pl.ds(s * rpt, rpt)])
    plsc.subcore_barrier()

    def chunk_body(j, _):
        pltpu.async_copy(hp_hbm.at[src_v.at[j]], rows_v, sem).wait()
        pltpu.sync_copy(rows_v, acc_sh.at[dst_v.at[j]], add=True)
        return 0

    lax.fori_loop(0, _NCHUNK, chunk_body, 0)
    plsc.subcore_barrier()
    pltpu.sync_copy(acc_sh.at[pl.ds(s * rpt, rpt)],
                    acc_hbm.at[c, pl.ds(s * rpt, rpt)])


_BLK = 256
_GRID = _NPAD // _BLK


def _tc_hp1_body(x_ref, d0_ref, d1_ref, w_ref, hp_ref, dinv_ref):
    deg = d0_ref[...] + d1_ref[...] + 1.0
    dinv = lax.rsqrt(deg)
    h = jnp.dot(x_ref[...], w_ref[...], preferred_element_type=jnp.float32)
    hp_ref[...] = h * dinv
    dinv_ref[...] = dinv


def _tc_hp1(x, d0, d1, w1t):
    return pl.pallas_call(
        _tc_hp1_body,
        grid=(_GRID,),
        in_specs=[
            pl.BlockSpec((_BLK, _D), lambda i: (i, 0)),
            pl.BlockSpec((_BLK, 1), lambda i: (i, 0)),
            pl.BlockSpec((_BLK, 1), lambda i: (i, 0)),
            pl.BlockSpec((_D, _D), lambda i: (0, 0)),
        ],
        out_specs=[
            pl.BlockSpec((_BLK, _D), lambda i: (i, 0)),
            pl.BlockSpec((_BLK, 1), lambda i: (i, 0)),
        ],
        out_shape=[
            jax.ShapeDtypeStruct((_NPAD, _D), jnp.float32),
            jax.ShapeDtypeStruct((_NPAD, 1), jnp.float32),
        ],
    )(x, d0, d1, w1t)


def _tc_hp2_body(p0_ref, p1_ref, hp_ref, dinv_ref, b1_ref, w_ref, out_ref):
    dinv = dinv_ref[...]
    x1 = dinv * (p0_ref[...] + p1_ref[...] + hp_ref[...]) + b1_ref[...]
    x1 = jnp.maximum(x1, 0.0)
    out_ref[...] = jnp.dot(x1, w_ref[...],
                           preferred_element_type=jnp.float32) * dinv


def _tc_hp2(p0, p1, hp1, dinv, b1, w2t):
    return pl.pallas_call(
        _tc_hp2_body,
        grid=(_GRID,),
        in_specs=[
            pl.BlockSpec((_BLK, _D), lambda i: (i, 0)),
            pl.BlockSpec((_BLK, _D), lambda i: (i, 0)),
            pl.BlockSpec((_BLK, _D), lambda i: (i, 0)),
            pl.BlockSpec((_BLK, 1), lambda i: (i, 0)),
            pl.BlockSpec((1, _D), lambda i: (0, 0)),
            pl.BlockSpec((_D, _D), lambda i: (0, 0)),
        ],
        out_specs=pl.BlockSpec((_BLK, _D), lambda i: (i, 0)),
        out_shape=jax.ShapeDtypeStruct((_NPAD, _D), jnp.float32),
    )(p0, p1, hp1, dinv, b1, w2t)


def _tc_wx_body(q0_ref, q1_ref, hp_ref, dinv_ref, b2_ref, sw_ref, out_ref):
    x2 = dinv_ref[...] * (q0_ref[...] + q1_ref[...] + hp_ref[...]) + b2_ref[...]
    out_ref[...] = x2 * sw_ref[...]


def _tc_wx(q0, q1, hp2, dinv, b2, sw):
    return pl.pallas_call(
        _tc_wx_body,
        grid=(_GRID,),
        in_specs=[
            pl.BlockSpec((_BLK, _D), lambda i: (i, 0)),
            pl.BlockSpec((_BLK, _D), lambda i: (i, 0)),
            pl.BlockSpec((_BLK, _D), lambda i: (i, 0)),
            pl.BlockSpec((_BLK, 1), lambda i: (i, 0)),
            pl.BlockSpec((1, _D), lambda i: (0, 0)),
            pl.BlockSpec((_BLK, 1), lambda i: (i, 0)),
        ],
        out_specs=pl.BlockSpec((_BLK, _D), lambda i: (i, 0)),
        out_shape=jax.ShapeDtypeStruct((_NPAD, _D), jnp.float32),
    )(q0, q1, hp2, dinv, b2, sw)


def _dotT(a, b):
    return lax.dot_general(a, b, (((0,), (0,)), ((), ())),
                           preferred_element_type=jnp.float32)


def _tail_body(wx_ref, batch_ref, wiht_ref, whht_ref, bi_ref,
               mwt_ref, mb_ref, p1t_ref, pb1_ref, p2t_ref, pb2_ref,
               p3t_ref, pb3_ref, out_ref):
    wx = wx_ref[...]
    bvec = batch_ref[...]
    cols = lax.broadcasted_iota(jnp.int32, (1, _B), 1)
    m = (bvec == cols).astype(jnp.float32)

    sto = _dotT(m, wx)

    h = jnp.zeros((_B, _H), jnp.float32)
    cstate = jnp.zeros((_B, _H), jnp.float32)
    q_star = jnp.zeros((_B, 2 * _H), jnp.float32)
    for _ in range(2):
        gates = (jnp.dot(q_star, wiht_ref[...], preferred_element_type=jnp.float32)
                 + jnp.dot(h, whht_ref[...], preferred_element_type=jnp.float32)
                 + bi_ref[...])
        ig = jax.nn.sigmoid(gates[:, 0:_H])
        fg = jax.nn.sigmoid(gates[:, _H:2 * _H])
        gg = jnp.tanh(gates[:, 2 * _H:3 * _H])
        og = jax.nn.sigmoid(gates[:, 3 * _H:4 * _H])
        cstate = fg * cstate + ig * gg
        h = og * jnp.tanh(cstate)
        q = h
        e = jnp.sum(wx * jnp.dot(m, q, preferred_element_type=jnp.float32),
                    axis=1, keepdims=True)
        mseg = jnp.max(jnp.where(m > 0, e, -1e30), axis=0, keepdims=True)
        ex = jnp.exp(e - jnp.dot(m, mseg.T, preferred_element_type=jnp.float32))
        den = _dotT(m, ex)
        a = ex / (jnp.dot(m, den, preferred_element_type=jnp.float32) + 1e-16)
        r = _dotT(m, a * wx)
        q_star = jnp.concatenate([q, r], axis=1)

    mean = jnp.dot(q_star, mwt_ref[...],
                   preferred_element_type=jnp.float32) + mb_ref[...] + sto
    norm = jnp.sqrt(jnp.sum(mean * mean, axis=1, keepdims=True))
    mean = mean / jnp.maximum(norm, 1e-12)

    h1 = jnp.dot(mean, p1t_ref[...], preferred_element_type=jnp.float32) + pb1_ref[...]
    h1 = jnp.where(h1 > 0, h1, 0.01 * h1)
    h2 = jnp.dot(h1, p2t_ref[...], preferred_element_type=jnp.float32) + pb2_ref[...]
    h2 = jnp.where(h2 > 0, h2, 0.01 * h2)
    out_ref[...] = jnp.dot(h2, p3t_ref[...],
                           preferred_element_type=jnp.float32) + pb3_ref[...]


def _tc_tail(wx, batch2, wiht, whht, bi, mwt, mb, p1t, pb1, p2t, pb2, p3t, pb3):
    return pl.pallas_call(
        _tail_body,
        out_shape=jax.ShapeDtypeStruct((_B, 1), jnp.float32),
    )(wx, batch2, wiht, whht, bi, mwt, mb, p1t, pb1, p2t, pb2, p3t, pb3)


def kernel(sto_x, sto_edge_index, sto_weight, sto_batch, weight,
           W1, b1, W2, b2, Wih, Whh, bih, bhh, mean_W, mean_b,
           P1, pb1, P2, pb2, P3, pb3):
    sto_x = sto_x.astype(jnp.int32)
    src = sto_edge_index[0].astype(jnp.int32)
    dst = sto_edge_index[1].astype(jnp.int32)
    batch = sto_batch.astype(jnp.int32)

    stox_p = jnp.concatenate(
        [sto_x, jnp.ones((_NPAD - _N,), jnp.int32)]).reshape(32, 4, 80)
    src_p = jnp.concatenate(
        [src, jnp.zeros((_EPAD - _E,), jnp.int32)]).reshape(2, 16, _NCHUNK, _EC)
    dst_p = jnp.concatenate(
        [dst, jnp.full((_EPAD - _E,), _N, jnp.int32)]).reshape(2, 16, _NCHUNK, _EC)
    dst_deg = dst.reshape(32, _DEG_PT)
    zeros1 = jnp.zeros((_NPAD,), jnp.float32)
    zeros2 = jnp.zeros((_NPAD, _D), jnp.float32)
    sw = jnp.concatenate(
        [sto_weight.astype(jnp.float32), jnp.zeros((_NPAD - _N,), jnp.float32)]
    ).reshape(_NPAD, 1)
    batch2 = jnp.concatenate(
        [batch, jnp.full((_NPAD - _N,), _B, jnp.int32)]).reshape(_NPAD, 1)

    w1t = W1.T
    w2t = W2.T
    b1r = b1.reshape(1, _D)
    b2r = b2.reshape(1, _D)
    wiht = Wih.T
    whht = Whh.T
    bi = (bih + bhh).reshape(1, 4 * _H)
    mwt = mean_W.T
    mb = mean_b.reshape(1, _H)
    p1t = P1.T
    pb1r = pb1.reshape(1, 128)
    p2t = P2.T
    pb2r = pb2.reshape(1, 128)
    p3t = P3.T
    pb3r = pb3.reshape(1, 1)

    x, degp = _sc_gather_deg(stox_p, dst_deg, zeros1, weight)
    d0 = degp[0].reshape(_NPAD, 1)
    d1 = degp[1].reshape(_NPAD, 1)

    hp1, dinv = _tc_hp1(x, d0, d1, w1t)
    acc1 = _sc_edge_pass(src_p, dst_p, zeros2, hp1)

    hp2 = _tc_hp2(acc1[0], acc1[1], hp1, dinv, b1r, w2t)
    acc2 = _sc_edge_pass(src_p, dst_p, zeros2, hp2)

    wx = _tc_wx(acc2[0], acc2[1], hp2, dinv, b2r, sw)
    return _tc_tail(wx, batch2, wiht, whht, bi, mwt, mb,
                    p1t, pb1r, p2t, pb2r, p3t, pb3r)


# speedup vs baseline: 9.0632x; 1.1735x over previous
import functools

import jax
import jax.numpy as jnp
from jax import lax
from jax.experimental import pallas as pl
from jax.experimental.pallas import tpu as pltpu
from jax.experimental.pallas import tpu_sc as plsc

_N = 10000
_E = 320000
_D = 128
_H = 128
_V = 10000
_B = 64

_NPAD = 10240
_ROWS_PER_TILE = _NPAD // 32
_EC = 64
_NCHUNK = 160
_EPT = _EC * _NCHUNK
_EPAD = _EPT * 32
_DEG_PT = _E // 32

_mesh = plsc.VectorSubcoreMesh(core_axis_name="c", subcore_axis_name="s")


@functools.partial(
    pl.kernel,
    out_type=(
        jax.ShapeDtypeStruct((_NPAD, _D), jnp.float32),
        jax.ShapeDtypeStruct((2, _NPAD), jnp.float32),
    ),
    mesh=_mesh,
    scratch_types=[
        pltpu.VMEM((4, 80), jnp.int32),
        pltpu.VMEM((_ROWS_PER_TILE, _D), jnp.float32),
        pltpu.VMEM((_DEG_PT,), jnp.int32),
        pltpu.VMEM((_NPAD,), jnp.float32),
        pltpu.VMEM((_NPAD // 16,), jnp.float32),
        pltpu.VMEM((_NPAD // 16,), jnp.float32),
        pltpu.VMEM_SHARED((16, _NPAD), jnp.float32),
        pltpu.SemaphoreType.DMA,
    ],
    compiler_params=pltpu.CompilerParams(needs_layout_passes=False),
)
def _sc_gather_deg(stox_hbm, dstd_hbm, zeros1_hbm, table_hbm,
                   x_hbm, deg_hbm,
                   idx_v, rows_v, dstl_v, degl_v, tmp_v, dacc_v, deg_st, sem):
    c = lax.axis_index("c")
    s = lax.axis_index("s")
    wid = c * 16 + s

    pltpu.sync_copy(stox_hbm.at[wid], idx_v)
    for r in range(4):
        for k in range(5):
            v = idx_v[r, pl.ds(k * 16, 16)]
            v = jnp.where(v == 0, _V - 1, v - 1)
            idx_v[r, pl.ds(k * 16, 16)] = v
    for r in range(4):
        pltpu.async_copy(table_hbm.at[idx_v.at[r]],
                         rows_v.at[pl.ds(r * 80, 80)], sem).wait()
    pltpu.sync_copy(rows_v, x_hbm.at[pl.ds(wid * _ROWS_PER_TILE, _ROWS_PER_TILE)])

    pltpu.sync_copy(dstd_hbm.at[wid], dstl_v)
    pltpu.sync_copy(zeros1_hbm, degl_v)
    ones = jnp.full((16,), 1.0, jnp.float32)

    def deg_body(k, _):
        dv = dstl_v[pl.ds(k * 16, 16)]
        plsc.addupdate_scatter(degl_v, [dv], ones)
        return 0

    lax.fori_loop(0, _DEG_PT // 16, deg_body, 0)

    pltpu.sync_copy(degl_v, deg_st.at[s])
    plsc.subcore_barrier()
    nsl = _NPAD // 16
    rslice = pl.ds(s * nsl, nsl)
    pltpu.sync_copy(deg_st.at[0, rslice], dacc_v)
    for slot in range(1, 16):
        pltpu.sync_copy(deg_st.at[slot, rslice], tmp_v)

        def add_body(k, _):
            dacc_v[pl.ds(k * 16, 16)] = (dacc_v[pl.ds(k * 16, 16)]
                                         + tmp_v[pl.ds(k * 16, 16)])
            return 0

        lax.fori_loop(0, nsl // 16, add_body, 0)
    pltpu.sync_copy(dacc_v, deg_hbm.at[c, rslice])


@functools.partial(
    pl.kernel,
    out_type=jax.ShapeDtypeStruct((2, _NPAD, _D), jnp.float32),
    mesh=_mesh,
    scratch_types=[
        pltpu.VMEM((8, _EC), jnp.int32),
        pltpu.VMEM((8, _EC), jnp.int32),
        pltpu.VMEM((4, _EC, _D), jnp.float32),
        pltpu.VMEM_SHARED((_NPAD, _D), jnp.float32),
        pltpu.SemaphoreType.DMA((4,)),
        pltpu.SemaphoreType.DMA((4,)),
        pltpu.SemaphoreType.DMA((8,)),
        pltpu.SemaphoreType.DMA((8,)),
    ],
    compiler_params=pltpu.CompilerParams(needs_layout_passes=False),
)
def _sc_edge_pass(src_hbm, dst_hbm, zeros2_hbm, hp_hbm,
                  acc_hbm,
                  src_v, dst_v, rows_v, acc_sh, gsem, ssem, sisem, disem):
    c = lax.axis_index("c")
    s = lax.axis_index("s")
    rpt = _NPAD // 16

    pltpu.sync_copy(zeros2_hbm.at[pl.ds(s * rpt, rpt)],
                    acc_sh.at[pl.ds(s * rpt, rpt)])
    plsc.subcore_barrier()

    def i_desc(j, q):
        return (pltpu.make_async_copy(src_hbm.at[c, s, j], src_v.at[q],
                                      sisem.at[q]),
                pltpu.make_async_copy(dst_hbm.at[c, s, j], dst_v.at[q],
                                      disem.at[q]))

    def g_desc(j, b, q):
        return pltpu.make_async_copy(hp_hbm.at[src_v.at[q]],
                                     rows_v.at[b], gsem.at[b])

    def s_desc(j, b, q):
        return pltpu.make_async_copy(rows_v.at[b],
                                     acc_sh.at[dst_v.at[q]], ssem.at[b])

    for q in range(4):
        a, d = i_desc(q, q)
        a.start(); d.start()
    for j in range(2):
        a, d = i_desc(j, j)
        a.wait(); d.wait()
        g_desc(j, j, j).start()

    def outer(jo, _):
        for u in range(8):
            j = jo * 8 + u
            b = u % 4
            q = u % 8
            g_desc(j, b, q).wait()
            s_desc(j, b, q).start(add=True)

            @pl.when(j + 4 < _NCHUNK)
            def _():
                a, d = i_desc(j + 4, (u + 4) % 8)
                a.start(); d.start()

            @pl.when(j >= 1)
            def _():
                s_desc(j - 1, (b + 3) % 4, (q + 7) % 8).wait()

            @pl.when(j + 2 < _NCHUNK)
            def _():
                a, d = i_desc(j + 2, (u + 2) % 8)
                a.wait(); d.wait()
                g_desc(j + 2, (b + 2) % 4, (u + 2) % 8).start()
        return 0

    lax.fori_loop(0, _NCHUNK // 8, outer, 0)
    s_desc(_NCHUNK - 1, (_NCHUNK - 1) % 4, (_NCHUNK - 1) % 8).wait()
    plsc.subcore_barrier()
    pltpu.sync_copy(acc_sh.at[pl.ds(s * rpt, rpt)],
                    acc_hbm.at[c, pl.ds(s * rpt, rpt)])


_BLK = 256
_GRID = _NPAD // _BLK


def _tc_hp1_body(x_ref, d0_ref, d1_ref, w_ref, hp_ref, dinv_ref):
    deg = d0_ref[...] + d1_ref[...] + 1.0
    dinv = 1.0 / jnp.sqrt(deg)
    h = jnp.dot(x_ref[...], w_ref[...], preferred_element_type=jnp.float32)
    hp_ref[...] = h * dinv
    dinv_ref[...] = dinv


def _tc_hp1(x, d0, d1, w1t):
    return pl.pallas_call(
        _tc_hp1_body,
        grid=(_GRID,),
        in_specs=[
            pl.BlockSpec((_BLK, _D), lambda i: (i, 0)),
            pl.BlockSpec((_BLK, 1), lambda i: (i, 0)),
            pl.BlockSpec((_BLK, 1), lambda i: (i, 0)),
            pl.BlockSpec((_D, _D), lambda i: (0, 0)),
        ],
        out_specs=[
            pl.BlockSpec((_BLK, _D), lambda i: (i, 0)),
            pl.BlockSpec((_BLK, 1), lambda i: (i, 0)),
        ],
        out_shape=[
            jax.ShapeDtypeStruct((_NPAD, _D), jnp.float32),
            jax.ShapeDtypeStruct((_NPAD, 1), jnp.float32),
        ],
    )(x, d0, d1, w1t)


def _tc_hp2_body(p0_ref, p1_ref, hp_ref, dinv_ref, b1_ref, w_ref, out_ref):
    dinv = dinv_ref[...]
    x1 = dinv * (p0_ref[...] + p1_ref[...] + hp_ref[...]) + b1_ref[...]
    x1 = jnp.maximum(x1, 0.0)
    out_ref[...] = jnp.dot(x1, w_ref[...],
                           preferred_element_type=jnp.float32) * dinv


def _tc_hp2(p0, p1, hp1, dinv, b1, w2t):
    return pl.pallas_call(
        _tc_hp2_body,
        grid=(_GRID,),
        in_specs=[
            pl.BlockSpec((_BLK, _D), lambda i: (i, 0)),
            pl.BlockSpec((_BLK, _D), lambda i: (i, 0)),
            pl.BlockSpec((_BLK, _D), lambda i: (i, 0)),
            pl.BlockSpec((_BLK, 1), lambda i: (i, 0)),
            pl.BlockSpec((1, _D), lambda i: (0, 0)),
            pl.BlockSpec((_D, _D), lambda i: (0, 0)),
        ],
        out_specs=pl.BlockSpec((_BLK, _D), lambda i: (i, 0)),
        out_shape=jax.ShapeDtypeStruct((_NPAD, _D), jnp.float32),
    )(p0, p1, hp1, dinv, b1, w2t)


def _tc_wx_body(q0_ref, q1_ref, hp_ref, dinv_ref, b2_ref, sw_ref, out_ref):
    x2 = dinv_ref[...] * (q0_ref[...] + q1_ref[...] + hp_ref[...]) + b2_ref[...]
    out_ref[...] = x2 * sw_ref[...]


def _tc_wx(q0, q1, hp2, dinv, b2, sw):
    return pl.pallas_call(
        _tc_wx_body,
        grid=(_GRID,),
        in_specs=[
            pl.BlockSpec((_BLK, _D), lambda i: (i, 0)),
            pl.BlockSpec((_BLK, _D), lambda i: (i, 0)),
            pl.BlockSpec((_BLK, _D), lambda i: (i, 0)),
            pl.BlockSpec((_BLK, 1), lambda i: (i, 0)),
            pl.BlockSpec((1, _D), lambda i: (0, 0)),
            pl.BlockSpec((_BLK, 1), lambda i: (i, 0)),
        ],
        out_specs=pl.BlockSpec((_BLK, _D), lambda i: (i, 0)),
        out_shape=jax.ShapeDtypeStruct((_NPAD, _D), jnp.float32),
    )(q0, q1, hp2, dinv, b2, sw)


def _dotT(a, b):
    return lax.dot_general(a, b, (((0,), (0,)), ((), ())),
                           preferred_element_type=jnp.float32,
                           precision=lax.Precision.HIGHEST)


def _dotH(a, b):
    return jnp.dot(a, b, preferred_element_type=jnp.float32,
                   precision=lax.Precision.HIGHEST)


def _tail_body(wx_ref, batch_ref, wiht_ref, whht_ref, bi_ref,
               mwt_ref, mb_ref, p1t_ref, pb1_ref, p2t_ref, pb2_ref,
               p3t_ref, pb3_ref, out_ref):
    wx = wx_ref[...]
    bvec = batch_ref[...]
    cols = lax.broadcasted_iota(jnp.int32, (1, _B), 1)
    m = (bvec == cols).astype(jnp.float32)

    sto = _dotT(m, wx)

    h = jnp.zeros((_B, _H), jnp.float32)
    cstate = jnp.zeros((_B, _H), jnp.float32)
    q_star = jnp.zeros((_B, 2 * _H), jnp.float32)
    for _ in range(2):
        gates = (jnp.dot(q_star, wiht_ref[...], preferred_element_type=jnp.float32)
                 + jnp.dot(h, whht_ref[...], preferred_element_type=jnp.float32)
                 + bi_ref[...])
        ig = jax.nn.sigmoid(gates[:, 0:_H])
        fg = jax.nn.sigmoid(gates[:, _H:2 * _H])
        gg = jnp.tanh(gates[:, 2 * _H:3 * _H])
        og = jax.nn.sigmoid(gates[:, 3 * _H:4 * _H])
        cstate = fg * cstate + ig * gg
        h = og * jnp.tanh(cstate)
        q = h
        e = jnp.sum(wx * _dotH(m, q),
                    axis=1, keepdims=True)
        mseg = jnp.max(jnp.where(m > 0, e, -1e30), axis=0, keepdims=True)
        ex = jnp.exp(e - _dotH(m, mseg.T))
        den = _dotT(m, ex)
        a = ex / (_dotH(m, den) + 1e-16)
        r = _dotT(m, a * wx)
        q_star = jnp.concatenate([q, r], axis=1)

    mean = jnp.dot(q_star, mwt_ref[...],
                   preferred_element_type=jnp.float32) + mb_ref[...] + sto
    norm = jnp.sqrt(jnp.sum(mean * mean, axis=1, keepdims=True))
    mean = mean / jnp.maximum(norm, 1e-12)

    h1 = jnp.dot(mean, p1t_ref[...], preferred_element_type=jnp.float32) + pb1_ref[...]
    h1 = jnp.where(h1 > 0, h1, 0.01 * h1)
    h2 = jnp.dot(h1, p2t_ref[...], preferred_element_type=jnp.float32) + pb2_ref[...]
    h2 = jnp.where(h2 > 0, h2, 0.01 * h2)
    out_ref[...] = jnp.dot(h2, p3t_ref[...],
                           preferred_element_type=jnp.float32) + pb3_ref[...]


def _tc_tail(wx, batch2, wiht, whht, bi, mwt, mb, p1t, pb1, p2t, pb2, p3t, pb3):
    return pl.pallas_call(
        _tail_body,
        out_shape=jax.ShapeDtypeStruct((_B, 1), jnp.float32),
    )(wx, batch2, wiht, whht, bi, mwt, mb, p1t, pb1, p2t, pb2, p3t, pb3)


def kernel(sto_x, sto_edge_index, sto_weight, sto_batch, weight,
           W1, b1, W2, b2, Wih, Whh, bih, bhh, mean_W, mean_b,
           P1, pb1, P2, pb2, P3, pb3):
    sto_x = sto_x.astype(jnp.int32)
    src = sto_edge_index[0].astype(jnp.int32)
    dst = sto_edge_index[1].astype(jnp.int32)
    batch = sto_batch.astype(jnp.int32)

    stox_p = jnp.concatenate(
        [sto_x, jnp.ones((_NPAD - _N,), jnp.int32)]).reshape(32, 4, 80)
    src_p = jnp.concatenate(
        [src, jnp.zeros((_EPAD - _E,), jnp.int32)]).reshape(2, 16, _NCHUNK, _EC)
    dst_p = jnp.concatenate(
        [dst, jnp.full((_EPAD - _E,), _N, jnp.int32)]).reshape(2, 16, _NCHUNK, _EC)
    dst_deg = dst.reshape(32, _DEG_PT)
    zeros1 = jnp.zeros((_NPAD,), jnp.float32)
    zeros2 = jnp.zeros((_NPAD, _D), jnp.float32)
    sw = jnp.concatenate(
        [sto_weight.astype(jnp.float32), jnp.zeros((_NPAD - _N,), jnp.float32)]
    ).reshape(_NPAD, 1)
    batch2 = jnp.concatenate(
        [batch, jnp.full((_NPAD - _N,), _B, jnp.int32)]).reshape(_NPAD, 1)

    w1t = W1.T
    w2t = W2.T
    b1r = b1.reshape(1, _D)
    b2r = b2.reshape(1, _D)
    wiht = Wih.T
    whht = Whh.T
    bi = (bih + bhh).reshape(1, 4 * _H)
    mwt = mean_W.T
    mb = mean_b.reshape(1, _H)
    p1t = P1.T
    pb1r = pb1.reshape(1, 128)
    p2t = P2.T
    pb2r = pb2.reshape(1, 128)
    p3t = P3.T
    pb3r = pb3.reshape(1, 1)

    x, degp = _sc_gather_deg(stox_p, dst_deg, zeros1, weight)
    d0 = degp[0].reshape(_NPAD, 1)
    d1 = degp[1].reshape(_NPAD, 1)

    hp1, dinv = _tc_hp1(x, d0, d1, w1t)
    acc1 = _sc_edge_pass(src_p, dst_p, zeros2, hp1)

    hp2 = _tc_hp2(acc1[0], acc1[1], hp1, dinv, b1r, w2t)
    acc2 = _sc_edge_pass(src_p, dst_p, zeros2, hp2)

    wx = _tc_wx(acc2[0], acc2[1], hp2, dinv, b2r, sw)
    return _tc_tail(wx, batch2, wiht, whht, bi, mwt, mb,
                    p1t, pb1r, p2t, pb2r, p3t, pb3r)
